# Initial kernel scaffold; baseline (speedup 1.0000x reference)
import functools

import jax
import jax.numpy as jnp
from jax import lax
from jax.experimental import pallas as pl
from jax.experimental.pallas import tpu as pltpu
from jax.experimental.pallas import tpu_sc as plsc

N = 10000
E = 320000
D = 128

NC = 2
NS = 16
CHUNK = 128
NPAD = 10240
EPAD = NC * NS * 80 * CHUNK
NCHUNKS = EPAD // (NC * NS * CHUNK)
RPT = NPAD // NS

_mesh = plsc.VectorSubcoreMesh(core_axis_name="c", subcore_axis_name="s")


@functools.partial(
    pl.kernel, mesh=_mesh,
    out_type=jax.ShapeDtypeStruct((NC, NPAD, 16), jnp.float32),
    scratch_types=[
        pltpu.VMEM((NCHUNKS, CHUNK), jnp.int32),
        pltpu.VMEM((CHUNK, 16), jnp.float32),
        pltpu.VMEM_SHARED((NPAD, 16), jnp.float32),
    ],
)
def _sc_degree(dst_hbm, ones_hbm, z16_hbm, out_hbm, didx_v, ones_v, acc_sh):
    cid = lax.axis_index("c")
    sid = lax.axis_index("s")
    pltpu.sync_copy(z16_hbm, acc_sh.at[pl.ds(sid * RPT, RPT)])
    pltpu.sync_copy(dst_hbm.at[cid, sid], didx_v)
    pltpu.sync_copy(ones_hbm, ones_v)
    plsc.subcore_barrier()

    def body(j, c):
        pltpu.sync_copy(ones_v, acc_sh.at[didx_v.at[j]], add=True)
        return c

    lax.fori_loop(0, NCHUNKS, body, 0)
    plsc.subcore_barrier()
    pltpu.sync_copy(acc_sh.at[pl.ds(sid * RPT, RPT)],
                    out_hbm.at[cid, pl.ds(sid * RPT, RPT)])


@functools.partial(
    pl.kernel, mesh=_mesh,
    out_type=jax.ShapeDtypeStruct((NC, NPAD, D), jnp.float32),
    scratch_types=[
        pltpu.VMEM((NCHUNKS, CHUNK), jnp.int32),
        pltpu.VMEM((NCHUNKS, CHUNK), jnp.int32),
        pltpu.VMEM((CHUNK, D), jnp.float32),
        pltpu.VMEM_SHARED((NPAD, D), jnp.float32),
        pltpu.SemaphoreType.DMA,
    ],
)
def _sc_aggregate(yw_hbm, src_hbm, dst_hbm, z_hbm, out_hbm,
                  sidx_v, didx_v, rows_v, acc_sh, gsem):
    cid = lax.axis_index("c")
    sid = lax.axis_index("s")
    pltpu.sync_copy(z_hbm, acc_sh.at[pl.ds(sid * RPT, RPT)])
    pltpu.sync_copy(src_hbm.at[cid, sid], sidx_v)
    pltpu.sync_copy(dst_hbm.at[cid, sid], didx_v)
    plsc.subcore_barrier()

    def body(j, c):
        pltpu.async_copy(yw_hbm.at[sidx_v.at[j]], rows_v, gsem).wait()
        pltpu.sync_copy(rows_v, acc_sh.at[didx_v.at[j]], add=True)
        return c

    lax.fori_loop(0, NCHUNKS, body, 0)
    plsc.subcore_barrier()
    pltpu.sync_copy(acc_sh.at[pl.ds(sid * RPT, RPT)],
                    out_hbm.at[cid, pl.ds(sid * RPT, RPT)])


def _dinv_from(deg_ref):
    d = deg_ref[...]
    deg = d[0, :N, 0:1] + d[1, :N, 0:1] + 1.0
    return lax.rsqrt(deg)


def _tc_yw0_body(x_ref, w_ref, deg_ref, o_ref):
    dinv = _dinv_from(deg_ref)
    xw = jnp.dot(x_ref[...], w_ref[...], preferred_element_type=jnp.float32)
    o_ref[...] = xw * dinv


def _bn_relu(out, g, beta):
    mean = jnp.mean(out, axis=0, keepdims=True)
    var = jnp.mean((out - mean) ** 2, axis=0, keepdims=True)
    return jnp.maximum((out - mean) * lax.rsqrt(var + 1e-5) * g + beta, 0.0)


def _tc_mid_body(acc_ref, yw_ref, deg_ref, g_ref, beta_ref, b_ref, w_ref, o_ref):
    dinv = _dinv_from(deg_ref)
    acc = acc_ref[0, :N, :] + acc_ref[1, :N, :] + yw_ref[...]
    out = acc * dinv + b_ref[...]
    h = _bn_relu(out, g_ref[...], beta_ref[...])
    o_ref[...] = jnp.dot(h, w_ref[...], preferred_element_type=jnp.float32) * dinv


def _tc_final_body(acc_ref, yw_ref, deg_ref, g_ref, beta_ref, b_ref,
                   lw1_ref, lb1_ref, lw2_ref, lb2_ref, lw3_ref, lb3_ref, o_ref):
    dinv = _dinv_from(deg_ref)
    acc = acc_ref[0, :N, :] + acc_ref[1, :N, :] + yw_ref[...]
    out = acc * dinv + b_ref[...]
    h = _bn_relu(out, g_ref[...], beta_ref[...])
    m = jnp.maximum(
        jnp.dot(h, lw1_ref[...], preferred_element_type=jnp.float32) + lb1_ref[...],
        0.0)
    m = jnp.maximum(
        jnp.dot(m, lw2_ref[...], preferred_element_type=jnp.float32) + lb2_ref[...],
        0.0)
    o_ref[...] = (
        jnp.dot(m, lw3_ref[...], preferred_element_type=jnp.float32) + lb3_ref[...])


_f32 = jnp.float32


def kernel(x, edge_index, W0, b0, g0, beta0, W1, b1, g1, beta1,
           lw1, lb1, lw2, lb2, lw3, lb3):
    x = x.astype(_f32)
    src = edge_index[0].astype(jnp.int32)
    dst = edge_index[1].astype(jnp.int32)
    npad = EPAD - E
    src3 = jnp.concatenate([src, jnp.zeros((npad,), jnp.int32)]
                           ).reshape(NC, NS, NCHUNKS, CHUNK)
    dst3 = jnp.concatenate([dst, jnp.full((npad,), N, jnp.int32)]
                           ).reshape(NC, NS, NCHUNKS, CHUNK)
    zrows = jnp.zeros((RPT, D), _f32)
    z16 = jnp.zeros((RPT, 16), _f32)
    ones16 = jnp.ones((CHUNK, 16), _f32)

    degp = _sc_degree(dst3, ones16, z16)

    b0r, g0r, beta0r = (v.reshape(1, D).astype(_f32) for v in (b0, g0, beta0))
    b1r, g1r, beta1r = (v.reshape(1, D).astype(_f32) for v in (b1, g1, beta1))
    lb1r, lb2r, lb3r = (v.reshape(1, D).astype(_f32) for v in (lb1, lb2, lb3))

    yw0 = pl.pallas_call(
        _tc_yw0_body,
        out_shape=jax.ShapeDtypeStruct((N, D), _f32),
    )(x, W0.astype(_f32), degp)

    acc0 = _sc_aggregate(yw0, src3, dst3, zrows)

    yw1 = pl.pallas_call(
        _tc_mid_body,
        out_shape=jax.ShapeDtypeStruct((N, D), _f32),
    )(acc0, yw0, degp, g0r, beta0r, b0r, W1.astype(_f32))

    acc1 = _sc_aggregate(yw1, src3, dst3, zrows)

    out = pl.pallas_call(
        _tc_final_body,
        out_shape=jax.ShapeDtypeStruct((N, D), _f32),
    )(acc1, yw1, degp, g1r, beta1r, b1r,
      lw1.astype(_f32), lb1r, lw2.astype(_f32), lb2r, lw3.astype(_f32), lb3r)
    return out


# speedup vs baseline: 8.7126x; 8.7126x over previous
import functools

import jax
import jax.numpy as jnp
from jax import lax
from jax.experimental import pallas as pl
from jax.experimental.pallas import tpu as pltpu
from jax.experimental.pallas import tpu_sc as plsc

N = 10000
E = 320000
D = 128

NC = 2
NS = 16
CHUNK = 128
NPAD = 10240
EPAD = NC * NS * 80 * CHUNK
NCHUNKS = EPAD // (NC * NS * CHUNK)
RPT = NPAD // NS

_mesh = plsc.VectorSubcoreMesh(core_axis_name="c", subcore_axis_name="s")


@functools.partial(
    pl.kernel, mesh=_mesh,
    out_type=jax.ShapeDtypeStruct((NC, NPAD, D), jnp.float32),
    scratch_types=[
        pltpu.VMEM((NCHUNKS, CHUNK), jnp.int32),
        pltpu.VMEM((CHUNK, D), jnp.float32),
        pltpu.VMEM_SHARED((NPAD, D), jnp.float32),
    ],
)
def _sc_degree(dst_hbm, ones_hbm, z_hbm, out_hbm, didx_v, ones_v, acc_sh):
    cid = lax.axis_index("c")
    sid = lax.axis_index("s")
    pltpu.sync_copy(z_hbm, acc_sh.at[pl.ds(sid * RPT, RPT)])
    pltpu.sync_copy(dst_hbm.at[cid, sid], didx_v)
    pltpu.sync_copy(ones_hbm, ones_v)
    plsc.subcore_barrier()

    def body(j, c):
        pltpu.sync_copy(ones_v, acc_sh.at[didx_v.at[j]], add=True)
        return c

    lax.fori_loop(0, NCHUNKS, body, 0)
    plsc.subcore_barrier()
    pltpu.sync_copy(acc_sh.at[pl.ds(sid * RPT, RPT)],
                    out_hbm.at[cid, pl.ds(sid * RPT, RPT)])


@functools.partial(
    pl.kernel, mesh=_mesh,
    out_type=jax.ShapeDtypeStruct((NC, NPAD, D), jnp.float32),
    scratch_types=[
        pltpu.VMEM((NCHUNKS, CHUNK), jnp.int32),
        pltpu.VMEM((NCHUNKS, CHUNK), jnp.int32),
        pltpu.VMEM((CHUNK, D), jnp.float32),
        pltpu.VMEM_SHARED((NPAD, D), jnp.float32),
        pltpu.SemaphoreType.DMA,
    ],
)
def _sc_aggregate(yw_hbm, src_hbm, dst_hbm, z_hbm, out_hbm,
                  sidx_v, didx_v, rows_v, acc_sh, gsem):
    cid = lax.axis_index("c")
    sid = lax.axis_index("s")
    pltpu.sync_copy(z_hbm, acc_sh.at[pl.ds(sid * RPT, RPT)])
    pltpu.sync_copy(src_hbm.at[cid, sid], sidx_v)
    pltpu.sync_copy(dst_hbm.at[cid, sid], didx_v)
    plsc.subcore_barrier()

    def body(j, c):
        pltpu.async_copy(yw_hbm.at[sidx_v.at[j]], rows_v, gsem).wait()
        pltpu.sync_copy(rows_v, acc_sh.at[didx_v.at[j]], add=True)
        return c

    lax.fori_loop(0, NCHUNKS, body, 0)
    plsc.subcore_barrier()
    pltpu.sync_copy(acc_sh.at[pl.ds(sid * RPT, RPT)],
                    out_hbm.at[cid, pl.ds(sid * RPT, RPT)])


def _dinv_from(deg_ref):
    d = deg_ref[...]
    deg = d[0, :, 0:1] + d[1, :, 0:1] + 1.0
    return lax.rsqrt(deg)


def _tc_yw0_body(x_ref, w_ref, deg_ref, o_ref):
    dinv = _dinv_from(deg_ref)
    xw = jnp.dot(x_ref[...], w_ref[...], preferred_element_type=jnp.float32)
    o_ref[...] = xw * dinv


def _bn_relu(out, g, beta):
    mean = jnp.mean(out, axis=0, keepdims=True)
    var = jnp.mean((out - mean) ** 2, axis=0, keepdims=True)
    return jnp.maximum((out - mean) * lax.rsqrt(var + 1e-5) * g + beta, 0.0)


def _tc_mid_body(acc_ref, yw_ref, deg_ref, g_ref, beta_ref, b_ref, w_ref, o_ref):
    dinv = _dinv_from(deg_ref)
    acc = acc_ref[0, :N, :] + acc_ref[1, :N, :] + yw_ref[...]
    out = acc * dinv + b_ref[...]
    h = _bn_relu(out, g_ref[...], beta_ref[...])
    o_ref[...] = jnp.dot(h, w_ref[...], preferred_element_type=jnp.float32) * dinv


def _tc_final_body(acc_ref, yw_ref, deg_ref, g_ref, beta_ref, b_ref,
                   lw1_ref, lb1_ref, lw2_ref, lb2_ref, lw3_ref, lb3_ref, o_ref):
    dinv = _dinv_from(deg_ref)
    acc = acc_ref[0, :N, :] + acc_ref[1, :N, :] + yw_ref[...]
    out = acc * dinv + b_ref[...]
    h = _bn_relu(out, g_ref[...], beta_ref[...])
    m = jnp.maximum(
        jnp.dot(h, lw1_ref[...], preferred_element_type=jnp.float32) + lb1_ref[...],
        0.0)
    m = jnp.maximum(
        jnp.dot(m, lw2_ref[...], preferred_element_type=jnp.float32) + lb2_ref[...],
        0.0)
    o_ref[...] = (
        jnp.dot(m, lw3_ref[...], preferred_element_type=jnp.float32) + lb3_ref[...])


_f32 = jnp.float32


def kernel(x, edge_index, W0, b0, g0, beta0, W1, b1, g1, beta1,
           lw1, lb1, lw2, lb2, lw3, lb3):
    x = x.astype(_f32)
    src = edge_index[0].astype(jnp.int32)
    dst = edge_index[1].astype(jnp.int32)
    npad = EPAD - E
    src3 = jnp.concatenate([src, jnp.zeros((npad,), jnp.int32)]
                           ).reshape(NC, NS, NCHUNKS, CHUNK)
    dst3 = jnp.concatenate([dst, jnp.full((npad,), N, jnp.int32)]
                           ).reshape(NC, NS, NCHUNKS, CHUNK)
    zrows = jnp.zeros((RPT, D), _f32)
    ones_rows = jnp.ones((CHUNK, D), _f32)

    degp = _sc_degree(dst3, ones_rows, zrows)
    degc = degp[:, :N, 0:8]

    b0r, g0r, beta0r = (v.reshape(1, D).astype(_f32) for v in (b0, g0, beta0))
    b1r, g1r, beta1r = (v.reshape(1, D).astype(_f32) for v in (b1, g1, beta1))
    lb1r, lb2r, lb3r = (v.reshape(1, D).astype(_f32) for v in (lb1, lb2, lb3))

    yw0 = pl.pallas_call(
        _tc_yw0_body,
        out_shape=jax.ShapeDtypeStruct((N, D), _f32),
    )(x, W0.astype(_f32), degc)

    acc0 = _sc_aggregate(yw0, src3, dst3, zrows)

    yw1 = pl.pallas_call(
        _tc_mid_body,
        out_shape=jax.ShapeDtypeStruct((N, D), _f32),
    )(acc0, yw0, degc, g0r, beta0r, b0r, W1.astype(_f32))

    acc1 = _sc_aggregate(yw1, src3, dst3, zrows)

    out = pl.pallas_call(
        _tc_final_body,
        out_shape=jax.ShapeDtypeStruct((N, D), _f32),
    )(acc1, yw1, degc, g1r, beta1r, b1r,
      lw1.astype(_f32), lb1r, lw2.astype(_f32), lb2r, lw3.astype(_f32), lb3r)
    return out


# speedup vs baseline: 9.9353x; 1.1403x over previous
import functools

import jax
import jax.numpy as jnp
from jax import lax
from jax.experimental import pallas as pl
from jax.experimental.pallas import tpu as pltpu
from jax.experimental.pallas import tpu_sc as plsc

N = 10000
E = 320000
D = 128

NC = 2
NS = 16
CHUNK = 128
NPAD = 10240
EPAD = NC * NS * 80 * CHUNK
NCHUNKS = EPAD // (NC * NS * CHUNK)
RPT = NPAD // NS

_mesh = plsc.VectorSubcoreMesh(core_axis_name="c", subcore_axis_name="s")


@functools.partial(
    pl.kernel, mesh=_mesh,
    out_type=jax.ShapeDtypeStruct((NC, NPAD, D), jnp.float32),
    scratch_types=[
        pltpu.VMEM((NCHUNKS, CHUNK), jnp.int32),
        pltpu.VMEM((CHUNK, D), jnp.float32),
        pltpu.VMEM_SHARED((NPAD, D), jnp.float32),
    ],
)
def _sc_degree(dst_hbm, ones_hbm, z_hbm, out_hbm, didx_v, ones_v, acc_sh):
    cid = lax.axis_index("c")
    sid = lax.axis_index("s")
    pltpu.sync_copy(z_hbm, acc_sh.at[pl.ds(sid * RPT, RPT)])
    pltpu.sync_copy(dst_hbm.at[cid, sid], didx_v)
    pltpu.sync_copy(ones_hbm, ones_v)
    plsc.subcore_barrier()

    def body(j, c):
        pltpu.sync_copy(ones_v, acc_sh.at[didx_v.at[j]], add=True)
        return c

    lax.fori_loop(0, NCHUNKS, body, 0)
    plsc.subcore_barrier()
    pltpu.sync_copy(acc_sh.at[pl.ds(sid * RPT, RPT)],
                    out_hbm.at[cid, pl.ds(sid * RPT, RPT)])


NBUF = 2


@functools.partial(
    pl.kernel, mesh=_mesh,
    out_type=jax.ShapeDtypeStruct((NC, NPAD, D), jnp.float32),
    scratch_types=[
        pltpu.VMEM((NBUF, CHUNK), jnp.int32),
        pltpu.VMEM((NCHUNKS, CHUNK), jnp.int32),
        pltpu.VMEM((NBUF, CHUNK, D), jnp.float32),
        pltpu.VMEM_SHARED((NPAD, D), jnp.float32),
    ] + [pltpu.SemaphoreType.DMA] * NBUF,
)
def _sc_aggregate(yw_hbm, src_hbm, dst_hbm, z_hbm, out_hbm,
                  sidx_v, didx_v, rows_v, acc_sh, *gsems):
    cid = lax.axis_index("c")
    sid = lax.axis_index("s")
    pltpu.sync_copy(z_hbm, acc_sh.at[pl.ds(sid * RPT, RPT)])
    pltpu.sync_copy(dst_hbm.at[cid, sid], didx_v)
    plsc.subcore_barrier()

    def _gather(b):
        bb = jnp.int32(b)
        return pltpu.make_async_copy(
            yw_hbm.at[sidx_v.at[bb]], rows_v.at[bb], gsems[b])

    def _stage_and_start(j, b):
        pltpu.sync_copy(src_hbm.at[cid, sid, j], sidx_v.at[jnp.int32(b)])
        _gather(b).start()

    for b in range(NBUF):
        _stage_and_start(jnp.int32(b), b)

    def outer(i, c):
        j0 = i * jnp.int32(NBUF)
        for b in range(NBUF):
            j = j0 + jnp.int32(b)
            _gather(b).wait()
            pltpu.sync_copy(rows_v.at[jnp.int32(b)], acc_sh.at[didx_v.at[j]],
                            add=True)
            jn = j + NBUF
            pl.when(jn < NCHUNKS)(lambda: _stage_and_start(jn, b))
        return c

    lax.fori_loop(jnp.int32(0), jnp.int32(NCHUNKS // NBUF), outer, 0)
    plsc.subcore_barrier()
    pltpu.sync_copy(acc_sh.at[pl.ds(sid * RPT, RPT)],
                    out_hbm.at[cid, pl.ds(sid * RPT, RPT)])


def _dinv_from(deg_ref):
    d = deg_ref[...]
    deg = d[0, :, 0:1] + d[1, :, 0:1] + 1.0
    return lax.rsqrt(deg)


def _tc_yw0_body(x_ref, w_ref, deg_ref, o_ref):
    dinv = _dinv_from(deg_ref)
    xw = jnp.dot(x_ref[...], w_ref[...], preferred_element_type=jnp.float32)
    o_ref[...] = xw * dinv


def _bn_relu(out, g, beta):
    mean = jnp.mean(out, axis=0, keepdims=True)
    var = jnp.mean((out - mean) ** 2, axis=0, keepdims=True)
    return jnp.maximum((out - mean) * lax.rsqrt(var + 1e-5) * g + beta, 0.0)


def _tc_mid_body(acc_ref, yw_ref, deg_ref, g_ref, beta_ref, b_ref, w_ref, o_ref):
    dinv = _dinv_from(deg_ref)
    acc = acc_ref[0, :N, :] + acc_ref[1, :N, :] + yw_ref[...]
    out = acc * dinv + b_ref[...]
    h = _bn_relu(out, g_ref[...], beta_ref[...])
    o_ref[...] = jnp.dot(h, w_ref[...], preferred_element_type=jnp.float32) * dinv


def _tc_final_body(acc_ref, yw_ref, deg_ref, g_ref, beta_ref, b_ref,
                   lw1_ref, lb1_ref, lw2_ref, lb2_ref, lw3_ref, lb3_ref, o_ref):
    dinv = _dinv_from(deg_ref)
    acc = acc_ref[0, :N, :] + acc_ref[1, :N, :] + yw_ref[...]
    out = acc * dinv + b_ref[...]
    h = _bn_relu(out, g_ref[...], beta_ref[...])
    m = jnp.maximum(
        jnp.dot(h, lw1_ref[...], preferred_element_type=jnp.float32) + lb1_ref[...],
        0.0)
    m = jnp.maximum(
        jnp.dot(m, lw2_ref[...], preferred_element_type=jnp.float32) + lb2_ref[...],
        0.0)
    o_ref[...] = (
        jnp.dot(m, lw3_ref[...], preferred_element_type=jnp.float32) + lb3_ref[...])


_f32 = jnp.float32


def kernel(x, edge_index, W0, b0, g0, beta0, W1, b1, g1, beta1,
           lw1, lb1, lw2, lb2, lw3, lb3):
    x = x.astype(_f32)
    src = edge_index[0].astype(jnp.int32)
    dst = edge_index[1].astype(jnp.int32)
    npad = EPAD - E
    src3 = jnp.concatenate([src, jnp.zeros((npad,), jnp.int32)]
                           ).reshape(NC, NS, NCHUNKS, CHUNK)
    dst3 = jnp.concatenate([dst, jnp.full((npad,), N, jnp.int32)]
                           ).reshape(NC, NS, NCHUNKS, CHUNK)
    zrows = jnp.zeros((RPT, D), _f32)
    ones_rows = jnp.ones((CHUNK, D), _f32)

    degp = _sc_degree(dst3, ones_rows, zrows)
    degc = degp[:, :N, 0:8]

    b0r, g0r, beta0r = (v.reshape(1, D).astype(_f32) for v in (b0, g0, beta0))
    b1r, g1r, beta1r = (v.reshape(1, D).astype(_f32) for v in (b1, g1, beta1))
    lb1r, lb2r, lb3r = (v.reshape(1, D).astype(_f32) for v in (lb1, lb2, lb3))

    yw0 = pl.pallas_call(
        _tc_yw0_body,
        out_shape=jax.ShapeDtypeStruct((N, D), _f32),
    )(x, W0.astype(_f32), degc)

    acc0 = _sc_aggregate(yw0, src3, dst3, zrows)

    yw1 = pl.pallas_call(
        _tc_mid_body,
        out_shape=jax.ShapeDtypeStruct((N, D), _f32),
    )(acc0, yw0, degc, g0r, beta0r, b0r, W1.astype(_f32))

    acc1 = _sc_aggregate(yw1, src3, dst3, zrows)

    out = pl.pallas_call(
        _tc_final_body,
        out_shape=jax.ShapeDtypeStruct((N, D), _f32),
    )(acc1, yw1, degc, g1r, beta1r, b1r,
      lw1.astype(_f32), lb1r, lw2.astype(_f32), lb2r, lw3.astype(_f32), lb3r)
    return out
